# baseline (speedup 1.0000x reference)
import functools

import jax
import jax.numpy as jnp
from jax import lax
from jax.experimental import pallas as pl
from jax.experimental.pallas import tpu as pltpu
from jax.experimental.pallas import tpu_sc as plsc


@functools.lru_cache(maxsize=None)
def _build(N, S, V, D):
    info = plsc.get_sparse_core_info()
    L = info.num_lanes
    NW = info.num_cores * info.num_subcores
    assert N % NW == 0
    BW = N // NW
    mesh = plsc.VectorSubcoreMesh(core_axis_name="c", subcore_axis_name="s")

    @functools.partial(
        pl.kernel,
        mesh=mesh,
        out_type=jax.ShapeDtypeStruct((S, D, N), jnp.float32),
        scratch_types=[
            pltpu.VMEM((S, BW), jnp.int32),
            pltpu.VMEM((BW, D), jnp.float32),
            pltpu.VMEM((BW, D), jnp.float32),
            pltpu.VMEM((D, BW), jnp.float32),
            pltpu.SemaphoreType.DMA,
        ],
        compiler_params=pltpu.CompilerParams(
            use_tc_tiling_on_sc=False, needs_layout_passes=False
        ),
    )
    def emb(xt_hbm, table_hbm, yt_hbm, idx_v, rows0_v, rows1_v, trans_v, sem):
        wid = lax.axis_index("s") * info.num_cores + lax.axis_index("c")
        b0 = wid * BW
        pltpu.sync_copy(xt_hbm.at[:, pl.ds(b0, BW)], idx_v)

        iota = lax.iota(jnp.int32, L)
        cols = [jnp.full((L,), d, jnp.int32) for d in range(D)]

        def gather(s, rows_v):
            pltpu.async_copy(table_hbm.at[idx_v.at[s]], rows_v, sem)

        def drain(rows_v):
            pltpu.make_async_copy(table_hbm.at[idx_v.at[0]], rows_v, sem).wait()

        def transpose_store(s, rows_v):
            def tbody(j0, carry):
                row_idx = iota + j0 * L
                for d in range(D):
                    trans_v[d, pl.ds(j0 * L, L)] = plsc.load_gather(
                        rows_v, [row_idx, cols[d]]
                    )
                return carry

            lax.fori_loop(0, BW // L, tbody, 0)
            pltpu.sync_copy(trans_v, yt_hbm.at[s].at[:, pl.ds(b0, BW)])

        gather(0, rows0_v)

        def body(i, carry):
            g = i * 2
            gather(g + 1, rows1_v)
            drain(rows0_v)
            transpose_store(g, rows0_v)

            @pl.when(g + 2 < S)
            def _():
                gather(g + 2, rows0_v)

            drain(rows1_v)
            transpose_store(g + 1, rows1_v)
            return carry

        lax.fori_loop(0, S // 2, body, 0)

    return emb


def kernel(x, table):
    N, S = x.shape
    V, D = table.shape
    yt = _build(N, S, V, D)(x.T, table)
    return yt.transpose(2, 0, 1)


# speedup vs baseline: 1.2989x; 1.2989x over previous
import functools

import jax
import jax.numpy as jnp
from jax import lax
from jax.experimental import pallas as pl
from jax.experimental.pallas import tpu as pltpu
from jax.experimental.pallas import tpu_sc as plsc


@functools.lru_cache(maxsize=None)
def _build(N, S, V, D):
    info = plsc.get_sparse_core_info()
    L = info.num_lanes
    NW = info.num_cores * info.num_subcores
    assert N % NW == 0
    BW = N // NW
    mesh = plsc.VectorSubcoreMesh(core_axis_name="c", subcore_axis_name="s")

    @functools.partial(
        pl.kernel,
        mesh=mesh,
        out_type=jax.ShapeDtypeStruct((S, D, N), jnp.float32),
        scratch_types=[
            pltpu.VMEM((S, BW), jnp.int32),
            pltpu.VMEM((BW, D), jnp.float32),
            pltpu.VMEM((BW, D), jnp.float32),
            pltpu.VMEM((D, BW), jnp.float32),
            pltpu.SemaphoreType.DMA,
        ],
        compiler_params=pltpu.CompilerParams(
            use_tc_tiling_on_sc=False, needs_layout_passes=False
        ),
    )
    def emb(xt_hbm, table_hbm, yt_hbm, idx_v, rows0_v, rows1_v, trans_v, sem):
        wid = lax.axis_index("s") * info.num_cores + lax.axis_index("c")
        b0 = wid * BW
        pltpu.sync_copy(xt_hbm.at[:, pl.ds(b0, BW)], idx_v)

        iota = lax.iota(jnp.int32, L)
        cols = [jnp.full((L,), d, jnp.int32) for d in range(D)]

        def gather(s, rows_v):
            pltpu.async_copy(table_hbm.at[idx_v.at[s]], rows_v, sem)

        def drain(rows_v):
            pltpu.make_async_copy(table_hbm.at[idx_v.at[0]], rows_v, sem).wait()

        def transpose_store(s, rows_v):
            @plsc.parallel_loop(0, BW // L, 1)
            def tbody(j0):
                row_idx = iota + j0 * L
                vals = [
                    plsc.load_gather(rows_v, [row_idx, cols[d]]) for d in range(D)
                ]
                for d in range(D):
                    trans_v[d, pl.ds(j0 * L, L)] = vals[d]

            pltpu.sync_copy(trans_v, yt_hbm.at[s].at[:, pl.ds(b0, BW)])

        gather(0, rows0_v)

        def body(i, carry):
            g = i * 2
            gather(g + 1, rows1_v)
            drain(rows0_v)
            transpose_store(g, rows0_v)

            @pl.when(g + 2 < S)
            def _():
                gather(g + 2, rows0_v)

            drain(rows1_v)
            transpose_store(g + 1, rows1_v)
            return carry

        lax.fori_loop(0, S // 2, body, 0)

    return emb


def kernel(x, table):
    N, S = x.shape
    V, D = table.shape
    yt = _build(N, S, V, D)(x.T, table)
    return yt.transpose(2, 0, 1)


# speedup vs baseline: 1.3088x; 1.0076x over previous
import functools

import jax
import jax.numpy as jnp
from jax import lax
from jax.experimental import pallas as pl
from jax.experimental.pallas import tpu as pltpu
from jax.experimental.pallas import tpu_sc as plsc


@functools.lru_cache(maxsize=None)
def _build(N, S, V, D):
    info = plsc.get_sparse_core_info()
    L = info.num_lanes
    NW = info.num_cores * info.num_subcores
    assert N % NW == 0
    BW = N // NW
    mesh = plsc.VectorSubcoreMesh(core_axis_name="c", subcore_axis_name="s")

    @functools.partial(
        pl.kernel,
        mesh=mesh,
        out_type=jax.ShapeDtypeStruct((S, D, N), jnp.float32),
        scratch_types=[
            pltpu.VMEM((S, BW), jnp.int32),
            pltpu.VMEM((BW, D), jnp.float32),
            pltpu.VMEM((BW, D), jnp.float32),
            pltpu.VMEM((D, BW), jnp.float32),
            pltpu.SemaphoreType.DMA,
        ],
        compiler_params=pltpu.CompilerParams(
            use_tc_tiling_on_sc=False, needs_layout_passes=False
        ),
    )
    def emb(xt_hbm, table_hbm, yt_hbm, idx_v, rows0_v, rows1_v, trans_v, sem):
        wid = lax.axis_index("s") * info.num_cores + lax.axis_index("c")
        b0 = wid * BW
        pltpu.sync_copy(xt_hbm.at[:, pl.ds(b0, BW)], idx_v)

        def scale_row(s, carry):
            @plsc.parallel_loop(0, BW // L, 1)
            def sbody(g):
                sl = (s, pl.ds(g * L, L))
                idx_v[sl] = idx_v[sl] * 4

            return carry

        lax.fori_loop(0, S, scale_row, 0)

        iota = lax.iota(jnp.int32, L)
        cols = [jnp.full((L,), d, jnp.int32) for d in range(D)]

        def gather(s, rows_v):
            pltpu.async_copy(table_hbm.at[idx_v.at[s]], rows_v, sem)

        def drain(rows_v):
            pltpu.make_async_copy(table_hbm.at[idx_v.at[0]], rows_v, sem).wait()

        def transpose_store(s, rows_v):
            @plsc.parallel_loop(0, BW // L, 1)
            def tbody(j0):
                row_idx = iota + j0 * L
                vals = [
                    plsc.load_gather(rows_v, [row_idx, cols[d]]) for d in range(D)
                ]
                for d in range(D):
                    trans_v[d, pl.ds(j0 * L, L)] = vals[d]

            pltpu.sync_copy(trans_v, yt_hbm.at[s].at[:, pl.ds(b0, BW)])

        gather(0, rows0_v)

        def body(i, carry):
            g = i * 2
            gather(g + 1, rows1_v)
            drain(rows0_v)
            transpose_store(g, rows0_v)

            @pl.when(g + 2 < S)
            def _():
                gather(g + 2, rows0_v)

            drain(rows1_v)
            transpose_store(g + 1, rows1_v)
            return carry

        lax.fori_loop(0, S // 2, body, 0)

    return emb


def kernel(x, table):
    N, S = x.shape
    V, D = table.shape
    pad = 128 // D * D - D if D < 128 else 0
    tablep = jnp.pad(table, ((0, 0), (0, pad))).reshape(-1, D)
    yt = _build(N, S, V, D)(x.T, tablep)
    return yt.transpose(2, 0, 1)


# speedup vs baseline: 1.3530x; 1.0338x over previous
import functools

import jax
import jax.numpy as jnp
from jax import lax
from jax.experimental import pallas as pl
from jax.experimental.pallas import tpu as pltpu
from jax.experimental.pallas import tpu_sc as plsc


@functools.lru_cache(maxsize=None)
def _build(N, S, V, D):
    info = plsc.get_sparse_core_info()
    L = info.num_lanes
    NW = info.num_cores * info.num_subcores
    assert N % NW == 0
    BW = N // NW
    mesh = plsc.VectorSubcoreMesh(core_axis_name="c", subcore_axis_name="s")

    @functools.partial(
        pl.kernel,
        mesh=mesh,
        out_type=jax.ShapeDtypeStruct((S, D, N), jnp.float32),
        scratch_types=[
            pltpu.VMEM((S, BW), jnp.int32),
            pltpu.VMEM((BW, D), jnp.float32),
            pltpu.VMEM((BW, D), jnp.float32),
            pltpu.VMEM((D, BW), jnp.float32),
            pltpu.VMEM((D, BW), jnp.float32),
            pltpu.SemaphoreType.DMA,
            pltpu.SemaphoreType.DMA,
        ],
        compiler_params=pltpu.CompilerParams(
            use_tc_tiling_on_sc=False, needs_layout_passes=False
        ),
    )
    def emb(
        xt_hbm, table_hbm, yt_hbm, idx_v, rows0_v, rows1_v, trans0_v, trans1_v,
        semg, sems,
    ):
        wid = lax.axis_index("s") * info.num_cores + lax.axis_index("c")
        b0 = wid * BW
        pltpu.sync_copy(xt_hbm.at[:, pl.ds(b0, BW)], idx_v)

        def scale_row(s, carry):
            @plsc.parallel_loop(0, BW // L, 1)
            def sbody(g):
                sl = (s, pl.ds(g * L, L))
                idx_v[sl] = idx_v[sl] * 4

            return carry

        lax.fori_loop(0, S, scale_row, 0)

        iota = lax.iota(jnp.int32, L)
        cols = [jnp.full((L,), d, jnp.int32) for d in range(D)]

        def gather(s, rows_v):
            pltpu.async_copy(table_hbm.at[idx_v.at[s]], rows_v, semg)

        def drain_gather(rows_v):
            pltpu.make_async_copy(table_hbm.at[idx_v.at[0]], rows_v, semg).wait()

        def drain_store(trans_v):
            pltpu.make_async_copy(
                trans_v, yt_hbm.at[0].at[:, pl.ds(b0, BW)], sems
            ).wait()

        def transpose(rows_v, trans_v):
            @plsc.parallel_loop(0, BW // L, 1)
            def tbody(j0):
                row_idx = iota + j0 * L
                vals = [
                    plsc.load_gather(rows_v, [row_idx, cols[d]]) for d in range(D)
                ]
                for d in range(D):
                    trans_v[d, pl.ds(j0 * L, L)] = vals[d]

        def store(s, trans_v):
            pltpu.async_copy(trans_v, yt_hbm.at[s].at[:, pl.ds(b0, BW)], sems)

        gather(0, rows0_v)

        def body(i, carry):
            g = i * 2
            gather(g + 1, rows1_v)
            drain_gather(rows0_v)

            @pl.when(g >= 2)
            def _():
                drain_store(trans0_v)

            transpose(rows0_v, trans0_v)
            store(g, trans0_v)

            @pl.when(g + 2 < S)
            def _():
                gather(g + 2, rows0_v)

            drain_gather(rows1_v)

            @pl.when(g >= 2)
            def _():
                drain_store(trans1_v)

            transpose(rows1_v, trans1_v)
            store(g + 1, trans1_v)
            return carry

        lax.fori_loop(0, S // 2, body, 0)
        drain_store(trans0_v)
        drain_store(trans1_v)

    return emb


def kernel(x, table):
    N, S = x.shape
    V, D = table.shape
    pad = 128 // D * D - D if D < 128 else 0
    tablep = jnp.pad(table, ((0, 0), (0, pad))).reshape(-1, D)
    yt = _build(N, S, V, D)(x.T, tablep)
    return yt.transpose(2, 0, 1)


# speedup vs baseline: 1.7968x; 1.3280x over previous
import functools

import jax
import jax.numpy as jnp
from jax import lax
from jax.experimental import pallas as pl
from jax.experimental.pallas import tpu as pltpu
from jax.experimental.pallas import tpu_sc as plsc


@functools.lru_cache(maxsize=None)
def _build(N, S, V, D):
    info = plsc.get_sparse_core_info()
    L = info.num_lanes
    NW = info.num_cores * info.num_subcores
    assert N % NW == 0
    BW = N // NW
    mesh = plsc.VectorSubcoreMesh(core_axis_name="c", subcore_axis_name="s")

    @functools.partial(
        pl.kernel,
        mesh=mesh,
        out_type=jax.ShapeDtypeStruct((S, D, N), jnp.float32),
        scratch_types=[
            pltpu.VMEM((S, BW), jnp.int32),
            pltpu.VMEM((BW, D), jnp.float32),
            pltpu.VMEM((BW, D), jnp.float32),
            pltpu.VMEM((D, BW), jnp.float32),
            pltpu.VMEM((D, BW), jnp.float32),
            pltpu.SemaphoreType.DMA,
            pltpu.SemaphoreType.DMA,
        ],
        compiler_params=pltpu.CompilerParams(
            use_tc_tiling_on_sc=False, needs_layout_passes=False
        ),
    )
    def emb(
        xt_hbm, table_hbm, yt_hbm, idx_v, rows0_v, rows1_v, trans0_v, trans1_v,
        semg, sems,
    ):
        wid = lax.axis_index("s") * info.num_cores + lax.axis_index("c")
        b0 = wid * BW
        pltpu.sync_copy(xt_hbm.at[:, pl.ds(b0, BW)], idx_v)

        def scale_row(s, carry):
            @plsc.parallel_loop(0, BW // L, 1)
            def sbody(g):
                sl = (s, pl.ds(g * L, L))
                idx_v[sl] = idx_v[sl] * 4

            return carry

        lax.fori_loop(0, S, scale_row, 0)

        iota = lax.iota(jnp.int32, L)
        rots = [(iota + c) & (L - 1) for c in range(L)]
        dcols = [iota + dt * L for dt in range(D // L)]

        def gather(s, rows_v):
            pltpu.async_copy(table_hbm.at[idx_v.at[s]], rows_v, semg)

        def drain_gather(rows_v):
            pltpu.make_async_copy(table_hbm.at[idx_v.at[0]], rows_v, semg).wait()

        def drain_store(trans_v):
            pltpu.make_async_copy(
                trans_v, yt_hbm.at[0].at[:, pl.ds(b0, BW)], sems
            ).wait()

        def transpose(rows_v, trans_v):
            @plsc.parallel_loop(0, BW // L, 1)
            def tbody(j0):
                base = j0 * L
                for c in range(L):
                    b_idx = rots[c] + base
                    for dcol in dcols:
                        v = plsc.load_gather(rows_v, [b_idx, dcol])
                        plsc.store_scatter(trans_v, [dcol, b_idx], v)

        def store(s, trans_v):
            pltpu.async_copy(trans_v, yt_hbm.at[s].at[:, pl.ds(b0, BW)], sems)

        gather(0, rows0_v)

        def body(i, carry):
            g = i * 2
            gather(g + 1, rows1_v)
            drain_gather(rows0_v)

            @pl.when(g >= 2)
            def _():
                drain_store(trans0_v)

            transpose(rows0_v, trans0_v)
            store(g, trans0_v)

            @pl.when(g + 2 < S)
            def _():
                gather(g + 2, rows0_v)

            drain_gather(rows1_v)

            @pl.when(g >= 2)
            def _():
                drain_store(trans1_v)

            transpose(rows1_v, trans1_v)
            store(g + 1, trans1_v)
            return carry

        lax.fori_loop(0, S // 2, body, 0)
        drain_store(trans0_v)
        drain_store(trans1_v)

    return emb


def kernel(x, table):
    N, S = x.shape
    V, D = table.shape
    pad = 128 // D * D - D if D < 128 else 0
    tablep = jnp.pad(table, ((0, 0), (0, pad))).reshape(-1, D)
    yt = _build(N, S, V, D)(x.T, tablep)
    return yt.transpose(2, 0, 1)


# speedup vs baseline: 2.1796x; 1.2130x over previous
import functools

import jax
import jax.numpy as jnp
from jax import lax
from jax.experimental import pallas as pl
from jax.experimental.pallas import tpu as pltpu
from jax.experimental.pallas import tpu_sc as plsc


@functools.lru_cache(maxsize=None)
def _build(N, S, V, D):
    info = plsc.get_sparse_core_info()
    L = info.num_lanes
    NW = info.num_cores * info.num_subcores
    assert N % NW == 0
    BW = N // NW
    mesh = plsc.VectorSubcoreMesh(core_axis_name="c", subcore_axis_name="s")

    @functools.partial(
        pl.kernel,
        mesh=mesh,
        out_type=jax.ShapeDtypeStruct((S, D // 8, N // 128, 1024), jnp.float32),
        scratch_types=[
            pltpu.VMEM((S, BW), jnp.int32),
            pltpu.VMEM((BW, D), jnp.float32),
            pltpu.VMEM((BW, D), jnp.float32),
            pltpu.VMEM((D // 8, BW // 128, 1024), jnp.float32),
            pltpu.VMEM((D // 8, BW // 128, 1024), jnp.float32),
            pltpu.SemaphoreType.DMA,
            pltpu.SemaphoreType.DMA,
        ],
        compiler_params=pltpu.CompilerParams(
            use_tc_tiling_on_sc=False, needs_layout_passes=False
        ),
    )
    def emb(
        xt_hbm, table_hbm, yt_hbm, idx_v, rows0_v, rows1_v, trans0_v, trans1_v,
        semg, sems,
    ):
        wid = lax.axis_index("s") * info.num_cores + lax.axis_index("c")
        b0 = wid * BW
        pltpu.sync_copy(xt_hbm.at[:, pl.ds(b0, BW)], idx_v)

        def scale_row(s, carry):
            @plsc.parallel_loop(0, BW // L, 1)
            def sbody(g):
                sl = (s, pl.ds(g * L, L))
                idx_v[sl] = idx_v[sl] * 4

            return carry

        lax.fori_loop(0, S, scale_row, 0)

        iota = lax.iota(jnp.int32, L)
        rots = [(iota + c) & (L - 1) for c in range(L)]
        dcols = [iota + t * L for t in range(D // L)]
        dtiles = [dcol >> 3 for dcol in dcols]
        dinners = [(dcol & 7) * 128 for dcol in dcols]

        def gather(s, rows_v):
            pltpu.async_copy(table_hbm.at[idx_v.at[s]], rows_v, semg)

        def drain_gather(rows_v):
            pltpu.make_async_copy(table_hbm.at[idx_v.at[0]], rows_v, semg).wait()

        bt0 = wid * (BW // 128)

        def drain_store(trans_v):
            pltpu.make_async_copy(
                trans_v, yt_hbm.at[0].at[:, pl.ds(bt0, BW // 128)], sems
            ).wait()

        def transpose(rows_v, trans_v):
            @plsc.parallel_loop(0, BW // L, 1)
            def tbody(j0):
                base = j0 * L
                for c in range(L):
                    b_idx = rots[c] + base
                    bt = b_idx >> 7
                    b128 = b_idx & 127
                    for t in range(D // L):
                        v = plsc.load_gather(rows_v, [b_idx, dcols[t]])
                        plsc.store_scatter(
                            trans_v, [dtiles[t], bt, dinners[t] + b128], v
                        )

        def store(s, trans_v):
            pltpu.async_copy(
                trans_v, yt_hbm.at[s].at[:, pl.ds(bt0, BW // 128)], sems
            )

        gather(0, rows0_v)

        def body(i, carry):
            g = i * 2
            gather(g + 1, rows1_v)
            drain_gather(rows0_v)

            @pl.when(g >= 2)
            def _():
                drain_store(trans0_v)

            transpose(rows0_v, trans0_v)
            store(g, trans0_v)

            @pl.when(g + 2 < S)
            def _():
                gather(g + 2, rows0_v)

            drain_gather(rows1_v)

            @pl.when(g >= 2)
            def _():
                drain_store(trans1_v)

            transpose(rows1_v, trans1_v)
            store(g + 1, trans1_v)
            return carry

        lax.fori_loop(0, S // 2, body, 0)
        drain_store(trans0_v)
        drain_store(trans1_v)

    return emb


def kernel(x, table):
    N, S = x.shape
    V, D = table.shape
    pad = 128 // D * D - D if D < 128 else 0
    tablep = jnp.pad(table, ((0, 0), (0, pad))).reshape(-1, D)
    yt4 = _build(N, S, V, D)(x.T, tablep)
    yt5 = yt4.reshape(S, D // 8, N // 128, 8, 128)
    return yt5.transpose(2, 4, 0, 1, 3).reshape(N, S, D)
